# baseline (speedup 1.0000x reference)
import functools

import jax
import jax.numpy as jnp
from jax import lax
from jax.experimental import pallas as pl
from jax.experimental.pallas import tpu as pltpu
from jax.experimental.pallas import tpu_sc as plsc

N = 1000000
DIM = 64
B = 16384
NC = 2
NS = 16
NW = NC * NS
BPW = B // NW
CHUNK = 128
NCH = BPW // CHUNK
LANES = 16


def _sc_body(idx_hbm, mu_hbm, lv_hbm, mu_out, lv_out, std_out,
             idx_v, mu_v, lv_v, std_v, sem_mu, sem_lv):
    wid = lax.axis_index("s") * NC + lax.axis_index("c")
    base = wid * BPW

    pltpu.sync_copy(idx_hbm.at[pl.ds(wid * NCH, NCH)], idx_v)

    waits = []
    for j in range(NCH):
        rows = pl.ds(j * CHUNK, CHUNK)
        waits.append(pltpu.async_copy(mu_hbm.at[idx_v.at[j]], mu_v.at[rows], sem_mu))
        waits.append(pltpu.async_copy(lv_hbm.at[idx_v.at[j]], lv_v.at[rows], sem_lv))
    for w in waits:
        w.wait()

    def row_body(i, carry):
        for c in range(DIM // LANES):
            cols = pl.ds(c * LANES, LANES)
            std_v[i, cols] = jnp.exp(lv_v[i, cols] * 0.5)
        return carry
    lax.fori_loop(0, BPW, row_body, 0, unroll=4)

    out_rows = pl.ds(base, BPW)
    pltpu.sync_copy(mu_v, mu_out.at[out_rows])
    pltpu.sync_copy(lv_v, lv_out.at[out_rows])
    pltpu.sync_copy(std_v, std_out.at[out_rows])


@jax.jit
def _vadlogvar_sc(idx2d, weight_mu, weight_logvar):
    out = jax.ShapeDtypeStruct((B, DIM), jnp.float32)
    mesh = plsc.VectorSubcoreMesh(core_axis_name="c", subcore_axis_name="s")
    run = functools.partial(
        pl.kernel,
        mesh=mesh,
        compiler_params=pltpu.CompilerParams(use_tc_tiling_on_sc=False),
        out_type=[out, out, out],
        scratch_types=[
            pltpu.VMEM((NCH, CHUNK), jnp.int32),
            pltpu.VMEM((BPW, DIM), jnp.float32),
            pltpu.VMEM((BPW, DIM), jnp.float32),
            pltpu.VMEM((BPW, DIM), jnp.float32),
            pltpu.SemaphoreType.DMA,
            pltpu.SemaphoreType.DMA,
        ],
    )(_sc_body)
    return run(idx2d, weight_mu, weight_logvar)


def kernel(idx, num_augment_pts, weight_mu, weight_logvar):
    del num_augment_pts
    idx2d = idx.astype(jnp.int32).reshape(B // CHUNK, CHUNK)
    mu, logvar, std = _vadlogvar_sc(idx2d, weight_mu, weight_logvar)
    return (mu, mu, logvar, std)


# speedup vs baseline: 1.2100x; 1.2100x over previous
import functools

import jax
import jax.numpy as jnp
from jax import lax
from jax.experimental import pallas as pl
from jax.experimental.pallas import tpu as pltpu
from jax.experimental.pallas import tpu_sc as plsc

N = 1000000
DIM = 64
B = 16384
NC = 2
NS = 16
NW = NC * NS
BPW = B // NW
LANES = 16
NG = BPW // LANES


def _sc_body(idx_hbm, mu_hbm, lv_hbm, mu_out, lv_out, std_out,
             idx_v, lv_rows, sem_mu, sem_lv):
    wid = lax.axis_index("s") * NC + lax.axis_index("c")
    base = wid * BPW

    pltpu.sync_copy(idx_hbm.at[pl.ds(wid * NG, NG)], idx_v)

    def fire_body(g, carry):
        vec = idx_v[g, pl.ds(0, LANES)]
        for l in range(LANES):
            row = vec[l]
            out_row = base + g * LANES + l
            loc_row = g * LANES + l
            pltpu.async_copy(mu_hbm.at[pl.ds(row, 1)],
                             mu_out.at[pl.ds(out_row, 1)], sem_mu)
            pltpu.async_copy(lv_hbm.at[pl.ds(row, 1)],
                             lv_rows.at[pl.ds(loc_row, 1)], sem_lv)
        return carry
    lax.fori_loop(0, NG, fire_body, 0)

    pltpu.make_async_copy(lv_hbm.at[pl.ds(0, BPW)], lv_rows, sem_lv).wait()

    out_rows = pl.ds(base, BPW)
    pltpu.sync_copy(lv_rows, lv_out.at[out_rows])

    def std_body(i, carry):
        for c in range(DIM // LANES):
            cols = pl.ds(c * LANES, LANES)
            lv_rows[i, cols] = jnp.exp(lv_rows[i, cols] * 0.5)
        return carry
    lax.fori_loop(0, BPW, std_body, 0, unroll=4)

    pltpu.sync_copy(lv_rows, std_out.at[out_rows])

    pltpu.make_async_copy(mu_hbm.at[pl.ds(0, BPW)],
                          mu_out.at[out_rows], sem_mu).wait()


@jax.jit
def _vadlogvar_sc(idx2d, weight_mu, weight_logvar):
    out = jax.ShapeDtypeStruct((B, DIM), jnp.float32)
    mesh = plsc.VectorSubcoreMesh(core_axis_name="c", subcore_axis_name="s")
    run = functools.partial(
        pl.kernel,
        mesh=mesh,
        out_type=[out, out, out],
        scratch_types=[
            pltpu.VMEM((NG, LANES), jnp.int32),
            pltpu.VMEM((BPW, DIM), jnp.float32),
            pltpu.SemaphoreType.DMA,
            pltpu.SemaphoreType.DMA,
        ],
    )(_sc_body)
    return run(idx2d, weight_mu, weight_logvar)


def kernel(idx, num_augment_pts, weight_mu, weight_logvar):
    del num_augment_pts
    idx2d = idx.astype(jnp.int32).reshape(B // LANES, LANES)
    mu, logvar, std = _vadlogvar_sc(idx2d, weight_mu, weight_logvar)
    return (mu, mu, logvar, std)
